# baseline (speedup 1.0000x reference)
import functools

import jax
import jax.numpy as jnp
from jax import lax
from jax.experimental import pallas as pl
from jax.experimental.pallas import tpu as pltpu
from jax.experimental.pallas import tpu_sc as plsc

_NUM_ITEM = 100000
_B = 1024
_L = 20
_H = 64

_NC = 2
_NS = 16
_NW = _NC * _NS
_RPW = _B // _NW
_WPW = _RPW * _NUM_ITEM
_CHUNKS_PW = _RPW * _L // 128
_CHUNKS_PAD = 8
_ZCHUNK = 64000
_ZDMAS = _WPW // _ZCHUNK


def _tc_probs_body(am_ref, lm_ref, seq_ref, wr_ref, ur_ref, vw_ref, vb_ref,
                   vals_ref, idx_ref):
    a = jnp.dot(
        am_ref[...].reshape(_B * _L, _H),
        ur_ref[...].T,
        preferred_element_type=jnp.float32,
        precision=lax.Precision.HIGHEST,
    )
    m = jnp.dot(
        lm_ref[...],
        wr_ref[...].T,
        preferred_element_type=jnp.float32,
        precision=lax.Precision.HIGHEST,
    )
    t = jnp.tanh(a.reshape(_B, _L, _H) + m[:, None, :])
    s = jnp.sum(t * vw_ref[...][None], axis=-1) + vb_ref[...]
    smax = jnp.max(s, axis=1, keepdims=True)
    e = jnp.exp(s - smax)
    p = e / jnp.sum(e, axis=1, keepdims=True)
    seq = seq_ref[...]
    acc = jnp.zeros_like(p)
    for l in range(_L):
        acc = acc + jnp.where(seq == seq[:, l:l + 1], p[:, l:l + 1], 0.0)
    vals_ref[...] = acc
    idx_ref[...] = seq + lax.broadcasted_iota(jnp.int32, (_B, _L), 0) * _NUM_ITEM


def _tc_probs(all_memory, last_memory, seq, Wr, Ur, Vr_w, Vr_b):
    return pl.pallas_call(
        _tc_probs_body,
        out_shape=[
            jax.ShapeDtypeStruct((_B, _L), jnp.float32),
            jax.ShapeDtypeStruct((_B, _L), jnp.int32),
        ],
    )(all_memory, last_memory, seq, Wr, Ur, Vr_w, Vr_b)


def _sc_scatter_body(idx_hbm, vals_hbm, zrow_hbm, out_hbm,
                     idx_v, vals_v, zrow_v, dma_sem, stage_sem):
    wid = lax.axis_index("s") * _NC + lax.axis_index("c")
    stage_z = pltpu.async_copy(zrow_hbm, zrow_v, stage_sem)
    stage_i = pltpu.async_copy(
        idx_hbm.at[pl.ds(wid * _CHUNKS_PAD, _CHUNKS_PAD)], idx_v, stage_sem)
    stage_v = pltpu.async_copy(
        vals_hbm.at[pl.ds(wid * _CHUNKS_PAD, _CHUNKS_PAD)], vals_v, stage_sem)
    out_base = pl.multiple_of(wid * _WPW, 128)
    stage_z.wait()
    zcopies = [
        pltpu.async_copy(
            zrow_v,
            out_hbm.at[pl.ds(out_base + r * _ZCHUNK, _ZCHUNK)],
            dma_sem,
        )
        for r in range(_ZDMAS)
    ]
    stage_i.wait()
    stage_v.wait()
    for cp in zcopies:
        cp.wait()
    scatters = [
        pltpu.async_copy(vals_v.at[j], out_hbm.at[idx_v.at[j]], dma_sem)
        for j in range(_CHUNKS_PW)
    ]
    for cp in scatters:
        cp.wait()


@functools.lru_cache(maxsize=1)
def _make_sc_scatter():
    return pl.kernel(
        _sc_scatter_body,
        out_type=jax.ShapeDtypeStruct((_B * _NUM_ITEM,), jnp.float32),
        mesh=plsc.VectorSubcoreMesh(core_axis_name="c", subcore_axis_name="s"),
        scratch_types=[
            pltpu.VMEM((_CHUNKS_PAD, 128), jnp.int32),
            pltpu.VMEM((_CHUNKS_PAD, 128), jnp.float32),
            pltpu.VMEM((_ZCHUNK,), jnp.float32),
            pltpu.SemaphoreType.DMA,
            pltpu.SemaphoreType.DMA,
        ],
    )


def _pad_chunks(x):
    x = x.reshape(_NW, _RPW * _L)
    x = jnp.pad(x, ((0, 0), (0, (_CHUNKS_PAD - _CHUNKS_PW) * 128)))
    return x.reshape(_NW * _CHUNKS_PAD, 128)


def kernel(all_memory, last_memory, seq_item, Wr, Ur, Vr_w, Vr_b):
    seq = seq_item.astype(jnp.int32)
    vals, idx = _tc_probs(all_memory, last_memory, seq, Wr, Ur, Vr_w, Vr_b)
    vals2 = _pad_chunks(vals)
    idx2 = _pad_chunks(idx)
    zrow = jnp.zeros((_ZCHUNK,), jnp.float32)
    out = _make_sc_scatter()(idx2, vals2, zrow)
    return out.reshape(_B, _NUM_ITEM)


# speedup vs baseline: 1.8695x; 1.8695x over previous
import functools

import jax
import jax.numpy as jnp
from jax import lax
from jax.experimental import pallas as pl
from jax.experimental.pallas import tpu as pltpu
from jax.experimental.pallas import tpu_sc as plsc

_NUM_ITEM = 100000
_B = 1024
_L = 20
_H = 64

_NC = 2
_NS = 16
_NW = _NC * _NS
_RPW = _B // _NW
_TR = _RPW // 8
_SPW = _RPW * _L
_G = 8 * _L // 16
_W = 6400
_KFULL = _NUM_ITEM // _W
_WLAST = _NUM_ITEM - _KFULL * _W


def _tc_probs_body(am_ref, lm_ref, seq_ref, wr_ref, ur_ref, vw_ref, vb_ref,
                   vals_ref, rows_ref, cols_ref):
    a = jnp.dot(
        am_ref[...].reshape(_B * _L, _H),
        ur_ref[...].T,
        preferred_element_type=jnp.float32,
        precision=lax.Precision.HIGHEST,
    )
    m = jnp.dot(
        lm_ref[...],
        wr_ref[...].T,
        preferred_element_type=jnp.float32,
        precision=lax.Precision.HIGHEST,
    )
    t = jnp.tanh(a.reshape(_B, _L, _H) + m[:, None, :])
    s = jnp.sum(t * vw_ref[...][None], axis=-1) + vb_ref[...]
    smax = jnp.max(s, axis=1, keepdims=True)
    e = jnp.exp(s - smax)
    p = e / jnp.sum(e, axis=1, keepdims=True)
    seq = seq_ref[...]
    acc = jnp.zeros_like(p)
    for l in range(_L):
        acc = acc + jnp.where(seq == seq[:, l:l + 1], p[:, l:l + 1], 0.0)
    vals_ref[...] = acc
    rows_ref[...] = lax.broadcasted_iota(jnp.int32, (_B, _L), 0) % 8
    cols_ref[...] = seq


def _tc_probs(all_memory, last_memory, seq, Wr, Ur, Vr_w, Vr_b):
    return pl.pallas_call(
        _tc_probs_body,
        out_shape=[
            jax.ShapeDtypeStruct((_B, _L), jnp.float32),
            jax.ShapeDtypeStruct((_B, _L), jnp.int32),
            jax.ShapeDtypeStruct((_B, _L), jnp.int32),
        ],
    )(all_memory, last_memory, seq, Wr, Ur, Vr_w, Vr_b)


def _sc_scatter_body(rows_hbm, cols_hbm, vals_hbm, zeros_hbm, zeros_tail_hbm,
                     out_hbm, rows_v, cols_v, vals_v, buf, tail_buf,
                     dma_sem, stage_sem):
    wid = lax.axis_index("s") * _NC + lax.axis_index("c")
    stage_r = pltpu.async_copy(
        rows_hbm.at[pl.ds(wid * _SPW, _SPW)], rows_v, stage_sem)
    stage_c = pltpu.async_copy(
        cols_hbm.at[pl.ds(wid * _SPW, _SPW)], cols_v, stage_sem)
    stage_v = pltpu.async_copy(
        vals_hbm.at[pl.ds(wid * _SPW, _SPW)], vals_v, stage_sem)
    stage_z = pltpu.async_copy(zeros_hbm, buf, stage_sem)
    stage_zt = pltpu.async_copy(zeros_tail_hbm, tail_buf, stage_sem)
    stage_r.wait()
    stage_c.wait()
    stage_v.wait()
    stage_z.wait()
    stage_zt.wait()

    def chunk_scatter(target, t, col_base, width, write_vals):
        for j in range(_G):
            o = (t * _G + j) * 16
            r16 = rows_v[pl.ds(o, 16)]
            c16 = cols_v[pl.ds(o, 16)] - col_base
            mask = (c16 >= 0) & (c16 < width)
            c16 = jnp.where(mask, c16, 0)
            x = vals_v[pl.ds(o, 16)] if write_vals else jnp.zeros(
                (16,), jnp.float32)
            plsc.store_scatter(target, [r16, c16], x, mask=mask)

    for t in range(_TR):
        row0 = pl.multiple_of((wid * _TR + t) * 8, 8)

        def body(k, _, t=t, row0=row0):
            col_base = pl.multiple_of(k * _W, 128)
            chunk_scatter(buf, t, col_base, _W, True)
            cp = pltpu.async_copy(
                buf,
                out_hbm.at[pl.ds(row0, 8), pl.ds(col_base, _W)],
                dma_sem,
            )
            cp.wait()
            chunk_scatter(buf, t, col_base, _W, False)
            return 0

        lax.fori_loop(0, _KFULL, body, 0)
        col_base = _KFULL * _W
        chunk_scatter(tail_buf, t, col_base, _WLAST, True)
        cp = pltpu.async_copy(
            tail_buf,
            out_hbm.at[pl.ds(row0, 8), pl.ds(col_base, _WLAST)],
            dma_sem,
        )
        cp.wait()
        chunk_scatter(tail_buf, t, col_base, _WLAST, False)


@functools.lru_cache(maxsize=1)
def _make_sc_scatter():
    return pl.kernel(
        _sc_scatter_body,
        out_type=jax.ShapeDtypeStruct((_B, _NUM_ITEM), jnp.float32),
        mesh=plsc.VectorSubcoreMesh(core_axis_name="c", subcore_axis_name="s"),
        compiler_params=pltpu.CompilerParams(needs_layout_passes=False),
        scratch_types=[
            pltpu.VMEM((_SPW,), jnp.int32),
            pltpu.VMEM((_SPW,), jnp.int32),
            pltpu.VMEM((_SPW,), jnp.float32),
            pltpu.VMEM((8, _W), jnp.float32),
            pltpu.VMEM((8, _WLAST), jnp.float32),
            pltpu.SemaphoreType.DMA,
            pltpu.SemaphoreType.DMA,
        ],
    )


def kernel(all_memory, last_memory, seq_item, Wr, Ur, Vr_w, Vr_b):
    seq = seq_item.astype(jnp.int32)
    vals, rows, cols = _tc_probs(all_memory, last_memory, seq, Wr, Ur,
                                 Vr_w, Vr_b)
    zeros = jnp.zeros((8, _W), jnp.float32)
    zeros_tail = jnp.zeros((8, _WLAST), jnp.float32)
    return _make_sc_scatter()(
        rows.reshape(-1), cols.reshape(-1), vals.reshape(-1), zeros,
        zeros_tail)
